# initial kernel scaffold (unmeasured)
import jax
import jax.numpy as jnp
from jax import lax
from jax.experimental import pallas as pl
from jax.experimental.pallas import tpu as pltpu


def kernel(
    x,
):
    def body(*refs):
        pass

    out_shape = jax.ShapeDtypeStruct(..., jnp.float32)
    return pl.pallas_call(body, out_shape=out_shape)(...)


# baseline (device time: 30734 ns/iter reference)
import jax
import jax.numpy as jnp
from jax import lax
from jax.experimental import pallas as pl
from jax.experimental.pallas import tpu as pltpu

N_DEV = 8


def kernel(x):
    m, n = x.shape

    def body(x_ref, out_ref, t_ref, comm_ref, send_sems, recv_sems):
        my_i = lax.axis_index("i")

        a = x_ref[...].astype(jnp.float32)
        d = 1
        while d < m:
            shifted = jnp.concatenate(
                [jnp.ones((d, n), jnp.float32), a[:-d, :]], axis=0
            )
            a = a * shifted
            d *= 2
        out_ref[...] = a
        t_ref[0, :] = a[m - 1, :]

        barrier_sem = pltpu.get_barrier_semaphore()
        for k in range(1, N_DEV):
            peer = (my_i + k) % N_DEV
            pl.semaphore_signal(
                barrier_sem,
                inc=1,
                device_id=(peer,),
                device_id_type=pl.DeviceIdType.MESH,
            )
        pl.semaphore_wait(barrier_sem, N_DEV - 1)

        rdmas = []
        for k in range(1, N_DEV):
            dst = (my_i + k) % N_DEV
            rdma = pltpu.make_async_remote_copy(
                src_ref=t_ref,
                dst_ref=comm_ref.at[pl.ds(k - 1, 1)],
                send_sem=send_sems.at[k - 1],
                recv_sem=recv_sems.at[k - 1],
                device_id=(dst,),
                device_id_type=pl.DeviceIdType.MESH,
            )
            rdma.start()
            rdmas.append(rdma)
        for rdma in rdmas:
            rdma.wait()

        comm = comm_ref[...]
        kvec = lax.broadcasted_iota(jnp.int32, (N_DEV - 1, n), 0) + 1
        vals = jnp.where(kvec <= my_i, comm, jnp.ones_like(comm))
        prefix = vals[0:1, :]
        for k in range(1, N_DEV - 1):
            prefix = prefix * vals[k : k + 1, :]
        out_ref[...] = out_ref[...] * prefix

    return pl.pallas_call(
        body,
        out_shape=jax.ShapeDtypeStruct((m, n), jnp.float32),
        in_specs=[pl.BlockSpec(memory_space=pltpu.VMEM)],
        out_specs=pl.BlockSpec(memory_space=pltpu.VMEM),
        scratch_shapes=[
            pltpu.VMEM((1, n), jnp.float32),
            pltpu.VMEM((N_DEV - 1, n), jnp.float32),
            pltpu.SemaphoreType.DMA((N_DEV - 1,)),
            pltpu.SemaphoreType.DMA((N_DEV - 1,)),
        ],
        compiler_params=pltpu.CompilerParams(collective_id=0),
    )(x)
